# baseline (speedup 1.0000x reference)
import functools

import jax
import jax.numpy as jnp
from jax import lax
from jax.experimental import pallas as pl
from jax.experimental.pallas import tpu as pltpu
from jax.experimental.pallas import tpu_sc as plsc

NC, NS, LANES = 2, 16, 16
NW = NC * NS
GCH = 128


def _make_sc_call(B, L, V):
    rows_per_w = B // NW
    nwords = rows_per_w * L
    nch = nwords // GCH
    nv = -(-L // LANES)
    rem = L - (nv - 1) * LANES

    def body(ids_hbm, w_hbm, b_hbm, out_hbm,
             ids_v, vals_v, buf_v, flat_v, out_v, b_v,
             w_sh, gsem, ssem, wsem):
        wid = lax.axis_index("s") * NC + lax.axis_index("c")
        base = wid * rows_per_w
        cpb = pltpu.async_copy(b_hbm, b_v, ssem)
        cpi = pltpu.async_copy(
            ids_hbm.at[pl.ds(pl.multiple_of(base * L, L), nwords)],
            ids_v, ssem)

        with jax.named_scope("stage"):
            @pl.when(lax.axis_index("s") == 0)
            def _():
                pltpu.async_copy(w_hbm.at[0], w_sh, wsem).wait()
            cpb.wait()
            cpi.wait()
            plsc.subcore_barrier()
        iota = lax.iota(jnp.int32, LANES)
        lastm = iota < rem
        bias = plsc.load_gather(b_v, [jnp.zeros((LANES,), jnp.int32)])

        with jax.named_scope("fire"):
            for c in range(nch):
                off = c * GCH
                pltpu.async_copy(
                    w_sh.at[ids_v.at[pl.ds(off, GCH)]],
                    vals_v.at[pl.ds(off, GCH)], gsem)

        def do_row(i, _):
            roff = pl.multiple_of(i * L, 8)
            pltpu.make_async_copy(
                out_hbm.at[pl.ds(0, L)],
                vals_v.at[pl.ds(roff, L)], gsem).wait()
            for k in range(nv - 1):
                idv = ids_v[pl.ds(roff + k * LANES, LANES)]
                plsc.store_scatter(buf_v, [idv], iota + (k * LANES))
            last_i = (i == rows_per_w - 1).astype(jnp.int32)
            shift = (LANES - rem) * last_i
            lo = pl.multiple_of(roff + (nv - 1) * LANES - shift, 8)
            lm = jnp.logical_xor(lastm, last_i > 0)
            pvec = iota + ((nv - 1) * LANES - shift)
            idl = ids_v[pl.ds(lo, LANES)]
            vll = vals_v[pl.ds(lo, LANES)]
            plsc.store_scatter(buf_v, [idl], pvec, mask=lm)
            acc = jnp.zeros((LANES,), jnp.float32)
            for k in range(nv - 1):
                idv = ids_v[pl.ds(roff + k * LANES, LANES)]
                back = plsc.load_gather(buf_v, [idv])
                keep = back == (iota + k * LANES)
                vals = vals_v[pl.ds(roff + k * LANES, LANES)]
                acc = acc + jnp.where(keep, vals, 0.0)
            backl = plsc.load_gather(buf_v, [idl])
            keepl = (backl == pvec) & lm
            acc = acc + jnp.where(keepl, vll, 0.0)
            foff = pl.multiple_of(lax.rem(i, LANES) * LANES, LANES)
            flat_v[pl.ds(foff, LANES)] = acc

            @pl.when(lax.rem(i, LANES) == LANES - 1)
            def _():
                tot = jnp.zeros((LANES,), jnp.float32)
                for l in range(LANES):
                    tot = tot + plsc.load_gather(flat_v, [iota * LANES + l])
                logits = tot + bias
                goff = pl.multiple_of(
                    (lax.div(i, LANES)) * LANES, LANES)
                out_v[pl.ds(goff, LANES)] = 1.0 / (1.0 + jnp.exp(-logits))
            return 0

        with jax.named_scope("rows"):
            lax.fori_loop(0, rows_per_w, do_row, 0)
        pltpu.sync_copy(
            out_v,
            out_hbm.at[pl.ds(pl.multiple_of(base, rows_per_w), rows_per_w)])

    call = functools.partial(
        pl.kernel,
        out_type=jax.ShapeDtypeStruct((B,), jnp.float32),
        mesh=plsc.VectorSubcoreMesh(
            core_axis_name="c", subcore_axis_name="s",
            num_cores=NC, num_subcores=NS),
        compiler_params=pltpu.CompilerParams(needs_layout_passes=False),
        scratch_types=[
            pltpu.VMEM((nwords,), jnp.int32),
            pltpu.VMEM((nwords,), jnp.float32),
            pltpu.VMEM((V,), jnp.int32),
            pltpu.VMEM((LANES * LANES,), jnp.float32),
            pltpu.VMEM((rows_per_w,), jnp.float32),
            pltpu.VMEM((1,), jnp.float32),
            pltpu.VMEM_SHARED((V,), jnp.float32),
            pltpu.SemaphoreType.DMA,
            pltpu.SemaphoreType.DMA,
            pltpu.SemaphoreType.DMA,
        ],
    )(body)
    return call


def kernel(token_ids, W, b):
    B, L = token_ids.shape
    V = W.shape[1]
    call = _make_sc_call(B, L, V)
    ids = token_ids.astype(jnp.int32).reshape(-1)
    out = call(ids, W.astype(jnp.float32), b.astype(jnp.float32))
    return out.reshape(B, 1)


# speedup vs baseline: 1.0078x; 1.0078x over previous
import functools

import jax
import jax.numpy as jnp
from jax import lax
from jax.experimental import pallas as pl
from jax.experimental.pallas import tpu as pltpu
from jax.experimental.pallas import tpu_sc as plsc

NC, NS, LANES = 2, 16, 16
NW = NC * NS
GCH = 128


def _make_sc_call(B, L, V):
    rows_per_w = B // NW
    nwords = rows_per_w * L
    nch = nwords // GCH
    nv = -(-L // LANES)
    rem = L - (nv - 1) * LANES

    def body(ids_hbm, w_hbm, b_hbm, out_hbm,
             ids_v, vals_v, buf_v, flat_v, out_v, b_v,
             w_sh, gsem, ssem, wsem):
        wid = lax.axis_index("s") * NC + lax.axis_index("c")
        base = wid * rows_per_w
        cpb = pltpu.async_copy(b_hbm, b_v, ssem)
        cpi = pltpu.async_copy(
            ids_hbm.at[pl.ds(pl.multiple_of(base * L, L), nwords)],
            ids_v, ssem)

        @pl.when(lax.axis_index("s") == 0)
        def _():
            pltpu.async_copy(w_hbm.at[0], w_sh, wsem).wait()
        cpb.wait()
        cpi.wait()
        plsc.subcore_barrier()
        iota = lax.iota(jnp.int32, LANES)
        lastm = iota < rem
        bias = plsc.load_gather(b_v, [jnp.zeros((LANES,), jnp.int32)])

        for c in range(nch):
            off = c * GCH
            pltpu.async_copy(
                w_sh.at[ids_v.at[pl.ds(off, GCH)]],
                vals_v.at[pl.ds(off, GCH)], gsem)

        def do_row(i, _):
            roff = pl.multiple_of(i * L, 8)
            pltpu.make_async_copy(
                out_hbm.at[pl.ds(0, L)],
                vals_v.at[pl.ds(roff, L)], gsem).wait()
            for k in range(nv - 1):
                idv = ids_v[pl.ds(roff + k * LANES, LANES)]
                plsc.store_scatter(buf_v, [idv], iota + (k * LANES))
            last_i = (i == rows_per_w - 1).astype(jnp.int32)
            shift = (LANES - rem) * last_i
            lo = pl.multiple_of(roff + (nv - 1) * LANES - shift, 8)
            lm = jnp.logical_xor(lastm, last_i > 0)
            pvec = iota + ((nv - 1) * LANES - shift)
            idl = ids_v[pl.ds(lo, LANES)]
            vll = vals_v[pl.ds(lo, LANES)]
            plsc.store_scatter(buf_v, [idl], pvec, mask=lm)
            acc = jnp.zeros((LANES,), jnp.float32)
            for k in range(nv - 1):
                idv = ids_v[pl.ds(roff + k * LANES, LANES)]
                back = plsc.load_gather(buf_v, [idv])
                keep = back == (iota + k * LANES)
                vals = vals_v[pl.ds(roff + k * LANES, LANES)]
                acc = acc + jnp.where(keep, vals, 0.0)
            backl = plsc.load_gather(buf_v, [idl])
            keepl = (backl == pvec) & lm
            acc = acc + jnp.where(keepl, vll, 0.0)
            foff = pl.multiple_of(lax.rem(i, LANES) * LANES, LANES)
            flat_v[pl.ds(foff, LANES)] = acc

            @pl.when(lax.rem(i, LANES) == LANES - 1)
            def _():
                tot = jnp.zeros((LANES,), jnp.float32)
                for l in range(LANES):
                    tot = tot + plsc.load_gather(flat_v, [iota * LANES + l])
                logits = tot + bias
                goff = pl.multiple_of(
                    (lax.div(i, LANES)) * LANES, LANES)
                out_v[pl.ds(goff, LANES)] = 1.0 / (1.0 + jnp.exp(-logits))
            return 0

        lax.fori_loop(0, rows_per_w, do_row, 0)
        pltpu.sync_copy(
            out_v,
            out_hbm.at[pl.ds(pl.multiple_of(base, rows_per_w), rows_per_w)])

    call = functools.partial(
        pl.kernel,
        out_type=jax.ShapeDtypeStruct((B,), jnp.float32),
        mesh=plsc.VectorSubcoreMesh(
            core_axis_name="c", subcore_axis_name="s",
            num_cores=NC, num_subcores=NS),
        compiler_params=pltpu.CompilerParams(needs_layout_passes=False),
        scratch_types=[
            pltpu.VMEM((nwords,), jnp.int32),
            pltpu.VMEM((nwords,), jnp.float32),
            pltpu.VMEM((V,), jnp.int32),
            pltpu.VMEM((LANES * LANES,), jnp.float32),
            pltpu.VMEM((rows_per_w,), jnp.float32),
            pltpu.VMEM((1,), jnp.float32),
            pltpu.VMEM_SHARED((V,), jnp.float32),
            pltpu.SemaphoreType.DMA,
            pltpu.SemaphoreType.DMA,
            pltpu.SemaphoreType.DMA,
        ],
    )(body)
    return call


def kernel(token_ids, W, b):
    B, L = token_ids.shape
    V = W.shape[1]
    call = _make_sc_call(B, L, V)
    ids = token_ids.astype(jnp.int32).reshape(-1)
    out = call(ids, W.astype(jnp.float32), b.astype(jnp.float32))
    return out.reshape(B, 1)
